# Initial kernel scaffold; baseline (speedup 1.0000x reference)
import jax
import jax.numpy as jnp
from jax.experimental import pallas as pl


def kernel(x, edge_index, batch, W1, b1, W2, b2, Wg, bg, Wf, bf, Wo, bo):
    raise NotImplementedError("write your pallas kernel here")


# speedup vs baseline: 13.2500x; 13.2500x over previous
import functools

import jax
import jax.numpy as jnp
from jax import lax
from jax.experimental import pallas as pl
from jax.experimental.pallas import tpu as pltpu
from jax.experimental.pallas import tpu_sc as plsc

N = 10000
NP = 10240
E = 640000
G = 128
F_IN = 114
H1 = 230
H1P = 256
H2 = 115
D = 128

NC = 2
NS = 16
NW = NC * NS
EPW = E // NW
CHUNK = 80
NCHUNK = EPW // CHUNK
RPS = NP // NS

ROWB = 640
GRID = NP // ROWB


def _sc_mesh():
    return plsc.VectorSubcoreMesh(core_axis_name="c", subcore_axis_name="s")


def _sc_degree(dst, zeros_np):
    @functools.partial(
        pl.kernel,
        out_type=jax.ShapeDtypeStruct((NC, NP), jnp.float32),
        mesh=_sc_mesh(),
        scratch_types=[
            pltpu.VMEM_SHARED((NP,), jnp.float32),
            pltpu.VMEM((CHUNK,), jnp.int32),
            pltpu.VMEM((CHUNK,), jnp.float32),
        ],
    )
    def deg_kernel(dst_hbm, zd_hbm, out_hbm, acc, dstv, onesv):
        c = lax.axis_index("c")
        s = lax.axis_index("s")
        pltpu.sync_copy(zd_hbm.at[pl.ds(s * RPS, RPS)],
                        acc.at[pl.ds(s * RPS, RPS)])
        for i in range(CHUNK // 16):
            onesv[pl.ds(i * 16, 16)] = jnp.ones((16,), jnp.float32)
        plsc.subcore_barrier()
        wbase = (c * NS + s) * EPW

        def body(k, carry):
            eb = wbase + k * CHUNK
            pltpu.sync_copy(dst_hbm.at[pl.ds(eb, CHUNK)], dstv)
            pltpu.sync_copy(onesv, acc.at[dstv], add=True)
            return carry

        lax.fori_loop(0, NCHUNK, body, 0)
        plsc.subcore_barrier()
        pltpu.sync_copy(acc.at[pl.ds(s * RPS, RPS)],
                        out_hbm.at[c, pl.ds(s * RPS, RPS)])

    return deg_kernel(dst, zeros_np)


def _sc_aggregate(table, src, dst, zeros_npd):
    @functools.partial(
        pl.kernel,
        out_type=jax.ShapeDtypeStruct((NC, NP, D), jnp.float32),
        mesh=_sc_mesh(),
        scratch_types=[
            pltpu.VMEM_SHARED((NP, D), jnp.float32),
            pltpu.VMEM((CHUNK,), jnp.int32),
            pltpu.VMEM((CHUNK,), jnp.int32),
            pltpu.VMEM((CHUNK, D), jnp.float32),
            pltpu.SemaphoreType.DMA,
        ],
    )
    def agg_kernel(table_hbm, src_hbm, dst_hbm, zt_hbm, out_hbm,
                   acc, srcv, dstv, rows, sem):
        c = lax.axis_index("c")
        s = lax.axis_index("s")
        pltpu.sync_copy(zt_hbm.at[pl.ds(s * RPS, RPS)],
                        acc.at[pl.ds(s * RPS, RPS)])
        plsc.subcore_barrier()
        wbase = (c * NS + s) * EPW

        def body(k, carry):
            eb = wbase + k * CHUNK
            pltpu.sync_copy(src_hbm.at[pl.ds(eb, CHUNK)], srcv)
            pltpu.sync_copy(dst_hbm.at[pl.ds(eb, CHUNK)], dstv)
            pltpu.async_copy(table_hbm.at[srcv], rows, sem).wait()
            pltpu.sync_copy(rows, acc.at[dstv], add=True)
            return carry

        lax.fori_loop(0, NCHUNK, body, 0)
        plsc.subcore_barrier()
        pltpu.sync_copy(acc.at[pl.ds(s * RPS, RPS)],
                        out_hbm.at[c, pl.ds(s * RPS, RPS)])

    return agg_kernel(table, src, dst, zeros_npd)


def _tc_scale(degT, xp):
    def body(deg_ref, x_ref, dinv_ref, xs_ref):
        d = 1.0 + deg_ref[:, 0:1] + deg_ref[:, 1:2]
        y = lax.rsqrt(d)
        dinv = y * (1.5 - 0.5 * d * y * y)
        dinv_ref[...] = dinv
        xs_ref[...] = x_ref[...] * dinv

    return pl.pallas_call(
        body,
        grid=(GRID,),
        in_specs=[
            pl.BlockSpec((ROWB, 2), lambda i: (i, 0)),
            pl.BlockSpec((ROWB, D), lambda i: (i, 0)),
        ],
        out_specs=[
            pl.BlockSpec((ROWB, 1), lambda i: (i, 0)),
            pl.BlockSpec((ROWB, D), lambda i: (i, 0)),
        ],
        out_shape=[
            jax.ShapeDtypeStruct((NP, 1), jnp.float32),
            jax.ShapeDtypeStruct((NP, D), jnp.float32),
        ],
    )(degT, xp)


def _tc_layers(xs, a0, a1, dinvc, W1p, b1p, W2p):
    def body(x_ref, a0_ref, a1_ref, dinv_ref, w1_ref, b1_ref, w2_ref, o_ref):
        dinv = dinv_ref[...]
        ax = dinv * (x_ref[...] + a0_ref[...] + a1_ref[...])
        h1 = jnp.dot(ax, w1_ref[...], preferred_element_type=jnp.float32,
                     precision=lax.Precision.HIGHEST)
        h1 = jnp.maximum(h1 + b1_ref[...], 0.0)
        m2 = jnp.dot(h1, w2_ref[...], preferred_element_type=jnp.float32,
                     precision=lax.Precision.HIGHEST)
        o_ref[...] = dinv * m2

    return pl.pallas_call(
        body,
        grid=(GRID,),
        in_specs=[
            pl.BlockSpec((ROWB, D), lambda i: (i, 0)),
            pl.BlockSpec((ROWB, D), lambda i: (i, 0)),
            pl.BlockSpec((ROWB, D), lambda i: (i, 0)),
            pl.BlockSpec((ROWB, 1), lambda i: (i, 0)),
            pl.BlockSpec((D, H1P), lambda i: (0, 0)),
            pl.BlockSpec((1, H1P), lambda i: (0, 0)),
            pl.BlockSpec((H1P, D), lambda i: (0, 0)),
        ],
        out_specs=pl.BlockSpec((ROWB, D), lambda i: (i, 0)),
        out_shape=jax.ShapeDtypeStruct((NP, D), jnp.float32),
    )(xs, a0, a1, dinvc, W1p, b1p, W2p)


def _tc_z(xs2, a0, a1, dinvc):
    def body(x_ref, a0_ref, a1_ref, dinv_ref, o_ref):
        o_ref[...] = dinv_ref[...] * (x_ref[...] + a0_ref[...] + a1_ref[...])

    return pl.pallas_call(
        body,
        grid=(GRID,),
        in_specs=[
            pl.BlockSpec((ROWB, D), lambda i: (i, 0)),
            pl.BlockSpec((ROWB, D), lambda i: (i, 0)),
            pl.BlockSpec((ROWB, D), lambda i: (i, 0)),
            pl.BlockSpec((ROWB, 1), lambda i: (i, 0)),
        ],
        out_specs=pl.BlockSpec((ROWB, D), lambda i: (i, 0)),
        out_shape=jax.ShapeDtypeStruct((NP, D), jnp.float32),
    )(xs2, a0, a1, dinvc)


def _tc_pool(z, batchc):
    def body(z_ref, b_ref, o_ref):
        s = pl.program_id(0)
        mask = b_ref[...] == s
        m = jnp.where(mask, z_ref[...], -jnp.inf)
        o_ref[0, ...] = jnp.max(m, axis=0, keepdims=True)

    return pl.pallas_call(
        body,
        grid=(G,),
        in_specs=[
            pl.BlockSpec((NP, D), lambda s: (0, 0)),
            pl.BlockSpec((NP, 1), lambda s: (0, 0)),
        ],
        out_specs=pl.BlockSpec((1, 1, D), lambda s: (s, 0, 0)),
        out_shape=jax.ShapeDtypeStruct((G, 1, D), jnp.float32),
    )(z, batchc).reshape(G, D)


def _tc_head(pooled, b2p, Wgp, bgp, Wfp, bfp, Wop, bop):
    def body(p_ref, b2_ref, wg_ref, bg_ref, wf_ref, bf_ref, wo_ref, bo_ref,
             o_ref):
        h = jnp.maximum(p_ref[...] + b2_ref[...], 0.0)
        g = jnp.dot(h, wg_ref[...], preferred_element_type=jnp.float32,
                     precision=lax.Precision.HIGHEST)
        g = jnp.maximum(g + bg_ref[...], 0.0)
        g = jnp.dot(g, wf_ref[...], preferred_element_type=jnp.float32,
                     precision=lax.Precision.HIGHEST)
        g = jnp.maximum(g + bf_ref[...], 0.0)
        o = jnp.dot(g, wo_ref[...], preferred_element_type=jnp.float32,
                     precision=lax.Precision.HIGHEST)
        o_ref[...] = o + bo_ref[...]

    return pl.pallas_call(
        body,
        out_shape=jax.ShapeDtypeStruct((G, D), jnp.float32),
    )(pooled, b2p, Wgp, bgp, Wfp, bfp, Wop, bop)


def kernel(x, edge_index, batch, W1, b1, W2, b2, Wg, bg, Wf, bf, Wo, bo):
    f32 = jnp.float32
    src = edge_index[0].astype(jnp.int32)
    dst = edge_index[1].astype(jnp.int32)

    xp = jnp.pad(x.astype(f32), ((0, NP - N), (0, D - F_IN)))
    zt = jnp.zeros((NP, D), f32)
    zd = jnp.zeros((NP,), f32)

    W1p = jnp.pad(W1, ((0, D - F_IN), (0, H1P - H1)))
    b1p = jnp.pad(b1, (0, H1P - H1)).reshape(1, H1P)
    W2p = jnp.pad(W2, ((0, H1P - H1), (0, D - H2)))
    b2p = jnp.pad(b2, (0, D - H2)).reshape(1, D)
    Wgp = jnp.pad(Wg, ((0, D - H2), (0, 128 - 64)))
    bgp = jnp.pad(bg, (0, 128 - 64)).reshape(1, 128)
    Wfp = jnp.pad(Wf, ((0, 128 - 64), (0, 128 - 32)))
    bfp = jnp.pad(bf, (0, 128 - 32)).reshape(1, 128)
    Wop = jnp.pad(Wo, ((0, 128 - 32), (0, 128 - 1)))
    bop = jnp.pad(bo, (0, 128 - 1)).reshape(1, 128)

    deg2 = _sc_degree(dst, zd)
    degT = jnp.transpose(deg2)
    dinvc, xs = _tc_scale(degT, xp)

    agg1 = _sc_aggregate(xs, src, dst, zt)
    xs2 = _tc_layers(xs, agg1[0], agg1[1], dinvc, W1p, b1p, W2p)

    agg2 = _sc_aggregate(xs2, src, dst, zt)
    z = _tc_z(xs2, agg2[0], agg2[1], dinvc)

    batchc = jnp.pad(batch.astype(jnp.int32), (0, NP - N),
                     constant_values=G).reshape(NP, 1)
    pooled = _tc_pool(z, batchc)
    outf = _tc_head(pooled, b2p, Wgp, bgp, Wfp, bfp, Wop, bop)
    return outf[:, :1]


# speedup vs baseline: 18.5779x; 1.4021x over previous
import functools

import jax
import jax.numpy as jnp
from jax import lax
from jax.experimental import pallas as pl
from jax.experimental.pallas import tpu as pltpu
from jax.experimental.pallas import tpu_sc as plsc

N = 10000
NP = 10240
E = 640000
G = 128
F_IN = 114
H1 = 230
H1P = 256
H2 = 115
D = 128

NC = 2
NS = 16
NW = NC * NS
EPW = E // NW
CHUNK = 80
NCHUNK = EPW // CHUNK
SEG = 50
NSEG = NCHUNK // SEG
RPS = NP // NS

ROWB = 640
GRID = NP // ROWB


def _sc_mesh():
    return plsc.VectorSubcoreMesh(core_axis_name="c", subcore_axis_name="s")


def _sc_degree(dst, zeros_np):
    @functools.partial(
        pl.kernel,
        out_type=jax.ShapeDtypeStruct((NC, NP), jnp.float32),
        mesh=_sc_mesh(),
        scratch_types=[
            pltpu.VMEM_SHARED((NP,), jnp.float32),
            pltpu.VMEM((CHUNK,), jnp.int32),
            pltpu.VMEM((CHUNK,), jnp.float32),
        ],
    )
    def deg_kernel(dst_hbm, zd_hbm, out_hbm, acc, dstv, onesv):
        c = lax.axis_index("c")
        s = lax.axis_index("s")
        pltpu.sync_copy(zd_hbm.at[pl.ds(s * RPS, RPS)],
                        acc.at[pl.ds(s * RPS, RPS)])
        for i in range(CHUNK // 16):
            onesv[pl.ds(i * 16, 16)] = jnp.ones((16,), jnp.float32)
        plsc.subcore_barrier()
        wbase = (c * NS + s) * EPW

        def body(k, carry):
            eb = wbase + k * CHUNK
            pltpu.sync_copy(dst_hbm.at[pl.ds(eb, CHUNK)], dstv)
            pltpu.sync_copy(onesv, acc.at[dstv], add=True)
            return carry

        lax.fori_loop(0, NCHUNK, body, 0)
        plsc.subcore_barrier()
        pltpu.sync_copy(acc.at[pl.ds(s * RPS, RPS)],
                        out_hbm.at[c, pl.ds(s * RPS, RPS)])

    return deg_kernel(dst, zeros_np)


def _sc_aggregate(table, src3, dst3, zeros_npd):
    @functools.partial(
        pl.kernel,
        out_type=jax.ShapeDtypeStruct((NC, NP, D), jnp.float32),
        mesh=_sc_mesh(),
        scratch_types=[
            pltpu.VMEM_SHARED((NP, D), jnp.float32),
            pltpu.VMEM((SEG, CHUNK), jnp.int32),
            pltpu.VMEM((SEG, CHUNK), jnp.int32),
            pltpu.VMEM((CHUNK, D), jnp.float32),
            pltpu.VMEM((CHUNK, D), jnp.float32),
            pltpu.SemaphoreType.DMA,
            pltpu.SemaphoreType.DMA,
        ],
    )
    def agg_kernel(table_hbm, src_hbm, dst_hbm, zt_hbm, out_hbm,
                   acc, srcv, dstv, rows0, rows1, sem0, sem1):
        c = lax.axis_index("c")
        s = lax.axis_index("s")
        w = c * NS + s
        pltpu.sync_copy(zt_hbm.at[pl.ds(s * RPS, RPS)],
                        acc.at[pl.ds(s * RPS, RPS)])
        plsc.subcore_barrier()

        def seg_body(g, carry):
            pltpu.sync_copy(src_hbm.at[w * NSEG + g], srcv)
            pltpu.sync_copy(dst_hbm.at[w * NSEG + g], dstv)
            pltpu.async_copy(table_hbm.at[srcv.at[0]], rows0, sem0)

            def body(k2, carry2):
                b0 = 2 * k2
                pltpu.make_async_copy(table_hbm.at[srcv.at[b0]],
                                      rows0, sem0).wait()
                pltpu.async_copy(table_hbm.at[srcv.at[b0 + 1]], rows1, sem1)
                pltpu.sync_copy(rows0, acc.at[dstv.at[b0]], add=True)

                @pl.when(k2 < SEG // 2 - 1)
                def _():
                    pltpu.async_copy(table_hbm.at[srcv.at[b0 + 2]],
                                     rows0, sem0)

                pltpu.make_async_copy(table_hbm.at[srcv.at[b0 + 1]],
                                      rows1, sem1).wait()
                pltpu.sync_copy(rows1, acc.at[dstv.at[b0 + 1]], add=True)
                return carry2

            lax.fori_loop(0, SEG // 2, body, 0)
            return carry

        lax.fori_loop(0, NSEG, seg_body, 0)
        plsc.subcore_barrier()
        pltpu.sync_copy(acc.at[pl.ds(s * RPS, RPS)],
                        out_hbm.at[c, pl.ds(s * RPS, RPS)])

    return agg_kernel(table, src3, dst3, zeros_npd)


def _tc_scale(degT):
    def body(deg_ref, dinv_ref):
        d = 1.0 + deg_ref[:, 0:1] + deg_ref[:, 1:2]
        y = lax.rsqrt(d)
        dinv = y * (1.5 - 0.5 * d * y * y)
        dinv_ref[...] = dinv

    return pl.pallas_call(
        body,
        grid=(GRID,),
        in_specs=[
            pl.BlockSpec((ROWB, 2), lambda i: (i, 0)),
        ],
        out_specs=pl.BlockSpec((ROWB, 1), lambda i: (i, 0)),
        out_shape=jax.ShapeDtypeStruct((NP, 1), jnp.float32),
    )(degT)


def _tc_mm1(xp, dinvc, W1p):
    def body(x_ref, dinv_ref, w1_ref, o_ref):
        h = jnp.dot(x_ref[...], w1_ref[...],
                    preferred_element_type=jnp.float32)
        o_ref[...] = dinv_ref[...] * h

    return pl.pallas_call(
        body,
        grid=(GRID,),
        in_specs=[
            pl.BlockSpec((ROWB, D), lambda i: (i, 0)),
            pl.BlockSpec((ROWB, 1), lambda i: (i, 0)),
            pl.BlockSpec((D, H1P), lambda i: (0, 0)),
        ],
        out_specs=pl.BlockSpec((ROWB, H1P), lambda i: (i, 0)),
        out_shape=jax.ShapeDtypeStruct((NP, H1P), jnp.float32),
    )(xp, dinvc, W1p)


def _tc_l1l2(hL, hR, aL0, aL1, aR0, aR1, dinvc, b1p, W2p):
    def body(hl_ref, hr_ref, al0_ref, al1_ref, ar0_ref, ar1_ref,
             dinv_ref, b1_ref, w2_ref, o_ref):
        dinv = dinv_ref[...]
        u = dinv * (hl_ref[...] + al0_ref[...] + al1_ref[...])
        v = dinv * (hr_ref[...] + ar0_ref[...] + ar1_ref[...])
        h1 = jnp.maximum(jnp.concatenate([u, v], axis=1) + b1_ref[...], 0.0)
        m2 = jnp.dot(h1, w2_ref[...], preferred_element_type=jnp.float32)
        o_ref[...] = dinv * m2

    return pl.pallas_call(
        body,
        grid=(GRID,),
        in_specs=[
            pl.BlockSpec((ROWB, D), lambda i: (i, 0)),
            pl.BlockSpec((ROWB, D), lambda i: (i, 0)),
            pl.BlockSpec((ROWB, D), lambda i: (i, 0)),
            pl.BlockSpec((ROWB, D), lambda i: (i, 0)),
            pl.BlockSpec((ROWB, D), lambda i: (i, 0)),
            pl.BlockSpec((ROWB, D), lambda i: (i, 0)),
            pl.BlockSpec((ROWB, 1), lambda i: (i, 0)),
            pl.BlockSpec((1, H1P), lambda i: (0, 0)),
            pl.BlockSpec((H1P, D), lambda i: (0, 0)),
        ],
        out_specs=pl.BlockSpec((ROWB, D), lambda i: (i, 0)),
        out_shape=jax.ShapeDtypeStruct((NP, D), jnp.float32),
    )(hL, hR, aL0, aL1, aR0, aR1, dinvc, b1p, W2p)


def _tc_z(xs2, a0, a1, dinvc):
    def body(x_ref, a0_ref, a1_ref, dinv_ref, o_ref):
        o_ref[...] = dinv_ref[...] * (x_ref[...] + a0_ref[...] + a1_ref[...])

    return pl.pallas_call(
        body,
        grid=(GRID,),
        in_specs=[
            pl.BlockSpec((ROWB, D), lambda i: (i, 0)),
            pl.BlockSpec((ROWB, D), lambda i: (i, 0)),
            pl.BlockSpec((ROWB, D), lambda i: (i, 0)),
            pl.BlockSpec((ROWB, 1), lambda i: (i, 0)),
        ],
        out_specs=pl.BlockSpec((ROWB, D), lambda i: (i, 0)),
        out_shape=jax.ShapeDtypeStruct((NP, D), jnp.float32),
    )(xs2, a0, a1, dinvc)


def _tc_pool(z, batchc):
    def body(z_ref, b_ref, o_ref):
        s = pl.program_id(0)
        mask = b_ref[...] == s
        m = jnp.where(mask, z_ref[...], -jnp.inf)
        o_ref[0, ...] = jnp.max(m, axis=0, keepdims=True)

    return pl.pallas_call(
        body,
        grid=(G,),
        in_specs=[
            pl.BlockSpec((NP, D), lambda s: (0, 0)),
            pl.BlockSpec((NP, 1), lambda s: (0, 0)),
        ],
        out_specs=pl.BlockSpec((1, 1, D), lambda s: (s, 0, 0)),
        out_shape=jax.ShapeDtypeStruct((G, 1, D), jnp.float32),
    )(z, batchc).reshape(G, D)


def _tc_head(pooled, b2p, Wgp, bgp, Wfp, bfp, Wop, bop):
    def body(p_ref, b2_ref, wg_ref, bg_ref, wf_ref, bf_ref, wo_ref, bo_ref,
             o_ref):
        h = jnp.maximum(p_ref[...] + b2_ref[...], 0.0)
        g = jnp.dot(h, wg_ref[...], preferred_element_type=jnp.float32)
        g = jnp.maximum(g + bg_ref[...], 0.0)
        g = jnp.dot(g, wf_ref[...], preferred_element_type=jnp.float32)
        g = jnp.maximum(g + bf_ref[...], 0.0)
        o = jnp.dot(g, wo_ref[...], preferred_element_type=jnp.float32)
        o_ref[...] = o + bo_ref[...]

    return pl.pallas_call(
        body,
        out_shape=jax.ShapeDtypeStruct((G, D), jnp.float32),
    )(pooled, b2p, Wgp, bgp, Wfp, bfp, Wop, bop)


def kernel(x, edge_index, batch, W1, b1, W2, b2, Wg, bg, Wf, bf, Wo, bo):
    f32 = jnp.float32
    src = edge_index[0].astype(jnp.int32)
    dst = edge_index[1].astype(jnp.int32)
    src3 = src.reshape(NW * NSEG, SEG, CHUNK)
    dst3 = dst.reshape(NW * NSEG, SEG, CHUNK)

    xp = jnp.pad(x.astype(f32), ((0, NP - N), (0, D - F_IN)))
    zt = jnp.zeros((NP, D), f32)
    zd = jnp.zeros((NP,), f32)

    W1p = jnp.pad(W1, ((0, D - F_IN), (0, H1P - H1)))
    b1p = jnp.pad(b1, (0, H1P - H1)).reshape(1, H1P)
    W2p = jnp.pad(W2, ((0, H1P - H1), (0, D - H2)))
    b2p = jnp.pad(b2, (0, D - H2)).reshape(1, D)
    Wgp = jnp.pad(Wg, ((0, D - H2), (0, 128 - 64)))
    bgp = jnp.pad(bg, (0, 128 - 64)).reshape(1, 128)
    Wfp = jnp.pad(Wf, ((0, 128 - 64), (0, 128 - 32)))
    bfp = jnp.pad(bf, (0, 128 - 32)).reshape(1, 128)
    Wop = jnp.pad(Wo, ((0, 128 - 32), (0, 128 - 1)))
    bop = jnp.pad(bo, (0, 128 - 1)).reshape(1, 128)

    deg2 = _sc_degree(dst, zd)
    degT = jnp.transpose(deg2)
    dinvc = _tc_scale(degT)

    hs = _tc_mm1(xp, dinvc, W1p)
    hL = hs[:, :D]
    hR = hs[:, D:]
    aggL = _sc_aggregate(hL, src3, dst3, zt)
    aggR = _sc_aggregate(hR, src3, dst3, zt)
    xs2 = _tc_l1l2(hL, hR, aggL[0], aggL[1], aggR[0], aggR[1],
                   dinvc, b1p, W2p)

    agg2 = _sc_aggregate(xs2, src3, dst3, zt)
    z = _tc_z(xs2, agg2[0], agg2[1], dinvc)

    batchc = jnp.pad(batch.astype(jnp.int32), (0, NP - N),
                     constant_values=G).reshape(NP, 1)
    pooled = _tc_pool(z, batchc)
    outf = _tc_head(pooled, b2p, Wgp, bgp, Wfp, bfp, Wop, bop)
    return outf[:, :1]


# speedup vs baseline: 22.0928x; 1.1892x over previous
import functools

import jax
import jax.numpy as jnp
from jax import lax
from jax.experimental import pallas as pl
from jax.experimental.pallas import tpu as pltpu
from jax.experimental.pallas import tpu_sc as plsc

N = 10000
NP = 10240
E = 640000
G = 128
F_IN = 114
H1 = 230
H1P = 256
H2 = 115
D = 128

NC = 2
NS = 16
NW = NC * NS
EPW = E // NW
CHUNK = 125
NCHUNK = EPW // CHUNK
SEG = 40
NSEG = NCHUNK // SEG
RPS = NP // NS

ROWB = 640
GRID = NP // ROWB


def _sc_mesh():
    return plsc.VectorSubcoreMesh(core_axis_name="c", subcore_axis_name="s")


def _sc_degree(dst3, zeros_np):
    @functools.partial(
        pl.kernel,
        out_type=jax.ShapeDtypeStruct((NC, NP), jnp.float32),
        mesh=_sc_mesh(),
        scratch_types=[
            pltpu.VMEM_SHARED((NP,), jnp.float32),
            pltpu.VMEM((SEG, CHUNK), jnp.int32),
            pltpu.VMEM((128,), jnp.float32),
            pltpu.SemaphoreType.DMA,
            pltpu.SemaphoreType.DMA,
        ],
    )
    def deg_kernel(dst_hbm, zd_hbm, out_hbm, acc, dstv, onesv, sem0, sem1):
        c = lax.axis_index("c")
        s = lax.axis_index("s")
        w = c * NS + s
        pltpu.sync_copy(zd_hbm.at[pl.ds(s * RPS, RPS)],
                        acc.at[pl.ds(s * RPS, RPS)])
        for i in range(128 // 16):
            onesv[pl.ds(i * 16, 16)] = jnp.ones((16,), jnp.float32)
        plsc.subcore_barrier()
        ones_c = onesv.at[pl.ds(0, CHUNK)]

        def seg_body(g, carry):
            pltpu.sync_copy(dst_hbm.at[w * NSEG + g], dstv)

            def body(k2, carry2):
                b0 = 2 * k2
                pltpu.async_copy(ones_c, acc.at[dstv.at[b0]], sem0,
                                 add=True)
                pltpu.async_copy(ones_c, acc.at[dstv.at[b0 + 1]], sem1,
                                 add=True)
                pltpu.make_async_copy(ones_c, acc.at[dstv.at[b0]],
                                      sem0).wait()
                pltpu.make_async_copy(ones_c, acc.at[dstv.at[b0 + 1]],
                                      sem1).wait()
                return carry2

            lax.fori_loop(0, SEG // 2, body, 0)
            return carry

        lax.fori_loop(0, NSEG, seg_body, 0)
        plsc.subcore_barrier()
        pltpu.sync_copy(acc.at[pl.ds(s * RPS, RPS)],
                        out_hbm.at[c, pl.ds(s * RPS, RPS)])

    return deg_kernel(dst3, zeros_np)


def _sc_aggregate(table, src3, dst3, zeros_npd):
    @functools.partial(
        pl.kernel,
        out_type=jax.ShapeDtypeStruct((NC, NP, D), jnp.float32),
        mesh=_sc_mesh(),
        scratch_types=[
            pltpu.VMEM_SHARED((NP, D), jnp.float32),
            pltpu.VMEM((SEG, CHUNK), jnp.int32),
            pltpu.VMEM((SEG, CHUNK), jnp.int32),
            pltpu.VMEM((CHUNK, D), jnp.float32),
            pltpu.VMEM((CHUNK, D), jnp.float32),
            pltpu.SemaphoreType.DMA,
            pltpu.SemaphoreType.DMA,
        ],
    )
    def agg_kernel(table_hbm, src_hbm, dst_hbm, zt_hbm, out_hbm,
                   acc, srcv, dstv, rows0, rows1, sem0, sem1):
        c = lax.axis_index("c")
        s = lax.axis_index("s")
        w = c * NS + s
        pltpu.sync_copy(zt_hbm.at[pl.ds(s * RPS, RPS)],
                        acc.at[pl.ds(s * RPS, RPS)])
        plsc.subcore_barrier()

        def seg_body(g, carry):
            pltpu.sync_copy(src_hbm.at[w * NSEG + g], srcv)
            pltpu.sync_copy(dst_hbm.at[w * NSEG + g], dstv)
            pltpu.async_copy(table_hbm.at[srcv.at[0]], rows0, sem0)

            def body(k2, carry2):
                b0 = 2 * k2
                pltpu.make_async_copy(table_hbm.at[srcv.at[b0]],
                                      rows0, sem0).wait()
                pltpu.async_copy(table_hbm.at[srcv.at[b0 + 1]], rows1, sem1)
                pltpu.sync_copy(rows0, acc.at[dstv.at[b0]], add=True)

                @pl.when(k2 < SEG // 2 - 1)
                def _():
                    pltpu.async_copy(table_hbm.at[srcv.at[b0 + 2]],
                                     rows0, sem0)

                pltpu.make_async_copy(table_hbm.at[srcv.at[b0 + 1]],
                                      rows1, sem1).wait()
                pltpu.sync_copy(rows1, acc.at[dstv.at[b0 + 1]], add=True)
                return carry2

            lax.fori_loop(0, SEG // 2, body, 0)
            return carry

        lax.fori_loop(0, NSEG, seg_body, 0)
        plsc.subcore_barrier()
        pltpu.sync_copy(acc.at[pl.ds(s * RPS, RPS)],
                        out_hbm.at[c, pl.ds(s * RPS, RPS)])

    return agg_kernel(table, src3, dst3, zeros_npd)


def _tc_scale(degT):
    def body(deg_ref, dinv_ref):
        d = 1.0 + deg_ref[:, 0:1] + deg_ref[:, 1:2]
        y = lax.rsqrt(d)
        dinv = y * (1.5 - 0.5 * d * y * y)
        dinv_ref[...] = dinv

    return pl.pallas_call(
        body,
        grid=(GRID,),
        in_specs=[
            pl.BlockSpec((ROWB, 2), lambda i: (i, 0)),
        ],
        out_specs=pl.BlockSpec((ROWB, 1), lambda i: (i, 0)),
        out_shape=jax.ShapeDtypeStruct((NP, 1), jnp.float32),
    )(degT)


def _tc_mm1(xp, dinvc, W1p):
    def body(x_ref, dinv_ref, w1_ref, o_ref):
        h = jnp.dot(x_ref[...], w1_ref[...],
                    preferred_element_type=jnp.float32)
        o_ref[...] = dinv_ref[...] * h

    return pl.pallas_call(
        body,
        grid=(GRID,),
        in_specs=[
            pl.BlockSpec((ROWB, D), lambda i: (i, 0)),
            pl.BlockSpec((ROWB, 1), lambda i: (i, 0)),
            pl.BlockSpec((D, H1P), lambda i: (0, 0)),
        ],
        out_specs=pl.BlockSpec((ROWB, H1P), lambda i: (i, 0)),
        out_shape=jax.ShapeDtypeStruct((NP, H1P), jnp.float32),
    )(xp, dinvc, W1p)


def _tc_l1l2(hL, hR, aL0, aL1, aR0, aR1, dinvc, b1p, W2p):
    def body(hl_ref, hr_ref, al0_ref, al1_ref, ar0_ref, ar1_ref,
             dinv_ref, b1_ref, w2_ref, o_ref):
        dinv = dinv_ref[...]
        u = dinv * (hl_ref[...] + al0_ref[...] + al1_ref[...])
        v = dinv * (hr_ref[...] + ar0_ref[...] + ar1_ref[...])
        h1 = jnp.maximum(jnp.concatenate([u, v], axis=1) + b1_ref[...], 0.0)
        m2 = jnp.dot(h1, w2_ref[...], preferred_element_type=jnp.float32)
        o_ref[...] = dinv * m2

    return pl.pallas_call(
        body,
        grid=(GRID,),
        in_specs=[
            pl.BlockSpec((ROWB, D), lambda i: (i, 0)),
            pl.BlockSpec((ROWB, D), lambda i: (i, 0)),
            pl.BlockSpec((ROWB, D), lambda i: (i, 0)),
            pl.BlockSpec((ROWB, D), lambda i: (i, 0)),
            pl.BlockSpec((ROWB, D), lambda i: (i, 0)),
            pl.BlockSpec((ROWB, D), lambda i: (i, 0)),
            pl.BlockSpec((ROWB, 1), lambda i: (i, 0)),
            pl.BlockSpec((1, H1P), lambda i: (0, 0)),
            pl.BlockSpec((H1P, D), lambda i: (0, 0)),
        ],
        out_specs=pl.BlockSpec((ROWB, D), lambda i: (i, 0)),
        out_shape=jax.ShapeDtypeStruct((NP, D), jnp.float32),
    )(hL, hR, aL0, aL1, aR0, aR1, dinvc, b1p, W2p)


def _tc_z(xs2, a0, a1, dinvc):
    def body(x_ref, a0_ref, a1_ref, dinv_ref, o_ref):
        o_ref[...] = dinv_ref[...] * (x_ref[...] + a0_ref[...] + a1_ref[...])

    return pl.pallas_call(
        body,
        grid=(GRID,),
        in_specs=[
            pl.BlockSpec((ROWB, D), lambda i: (i, 0)),
            pl.BlockSpec((ROWB, D), lambda i: (i, 0)),
            pl.BlockSpec((ROWB, D), lambda i: (i, 0)),
            pl.BlockSpec((ROWB, 1), lambda i: (i, 0)),
        ],
        out_specs=pl.BlockSpec((ROWB, D), lambda i: (i, 0)),
        out_shape=jax.ShapeDtypeStruct((NP, D), jnp.float32),
    )(xs2, a0, a1, dinvc)


def _tc_pool(z, batchc):
    def body(z_ref, b_ref, o_ref):
        s = pl.program_id(0)
        mask = b_ref[...] == s
        m = jnp.where(mask, z_ref[...], -jnp.inf)
        o_ref[0, ...] = jnp.max(m, axis=0, keepdims=True)

    return pl.pallas_call(
        body,
        grid=(G,),
        in_specs=[
            pl.BlockSpec((NP, D), lambda s: (0, 0)),
            pl.BlockSpec((NP, 1), lambda s: (0, 0)),
        ],
        out_specs=pl.BlockSpec((1, 1, D), lambda s: (s, 0, 0)),
        out_shape=jax.ShapeDtypeStruct((G, 1, D), jnp.float32),
    )(z, batchc).reshape(G, D)


def _tc_head(pooled, b2p, Wgp, bgp, Wfp, bfp, Wop, bop):
    def body(p_ref, b2_ref, wg_ref, bg_ref, wf_ref, bf_ref, wo_ref, bo_ref,
             o_ref):
        h = jnp.maximum(p_ref[...] + b2_ref[...], 0.0)
        g = jnp.dot(h, wg_ref[...], preferred_element_type=jnp.float32)
        g = jnp.maximum(g + bg_ref[...], 0.0)
        g = jnp.dot(g, wf_ref[...], preferred_element_type=jnp.float32)
        g = jnp.maximum(g + bf_ref[...], 0.0)
        o = jnp.dot(g, wo_ref[...], preferred_element_type=jnp.float32)
        o_ref[...] = o + bo_ref[...]

    return pl.pallas_call(
        body,
        out_shape=jax.ShapeDtypeStruct((G, D), jnp.float32),
    )(pooled, b2p, Wgp, bgp, Wfp, bfp, Wop, bop)


def kernel(x, edge_index, batch, W1, b1, W2, b2, Wg, bg, Wf, bf, Wo, bo):
    f32 = jnp.float32
    src = edge_index[0].astype(jnp.int32)
    dst = edge_index[1].astype(jnp.int32)
    src3 = src.reshape(NW * NSEG, SEG, CHUNK)
    dst3 = dst.reshape(NW * NSEG, SEG, CHUNK)

    xp = jnp.pad(x.astype(f32), ((0, NP - N), (0, D - F_IN)))
    zt = jnp.zeros((NP, D), f32)
    zd = jnp.zeros((NP,), f32)

    W1p = jnp.pad(W1, ((0, D - F_IN), (0, H1P - H1)))
    b1p = jnp.pad(b1, (0, H1P - H1)).reshape(1, H1P)
    W2p = jnp.pad(W2, ((0, H1P - H1), (0, D - H2)))
    b2p = jnp.pad(b2, (0, D - H2)).reshape(1, D)
    Wgp = jnp.pad(Wg, ((0, D - H2), (0, 128 - 64)))
    bgp = jnp.pad(bg, (0, 128 - 64)).reshape(1, 128)
    Wfp = jnp.pad(Wf, ((0, 128 - 64), (0, 128 - 32)))
    bfp = jnp.pad(bf, (0, 128 - 32)).reshape(1, 128)
    Wop = jnp.pad(Wo, ((0, 128 - 32), (0, 128 - 1)))
    bop = jnp.pad(bo, (0, 128 - 1)).reshape(1, 128)

    deg2 = _sc_degree(dst3, zd)
    degT = jnp.transpose(deg2)
    dinvc = _tc_scale(degT)

    hs = _tc_mm1(xp, dinvc, W1p)
    hL = hs[:, :D]
    hR = hs[:, D:]
    aggL = _sc_aggregate(hL, src3, dst3, zt)
    aggR = _sc_aggregate(hR, src3, dst3, zt)
    xs2 = _tc_l1l2(hL, hR, aggL[0], aggL[1], aggR[0], aggR[1],
                   dinvc, b1p, W2p)

    agg2 = _sc_aggregate(xs2, src3, dst3, zt)
    z = _tc_z(xs2, agg2[0], agg2[1], dinvc)

    batchc = jnp.pad(batch.astype(jnp.int32), (0, NP - N),
                     constant_values=G).reshape(NP, 1)
    pooled = _tc_pool(z, batchc)
    outf = _tc_head(pooled, b2p, Wgp, bgp, Wfp, bfp, Wop, bop)
    return outf[:, :1]


# speedup vs baseline: 26.4500x; 1.1972x over previous
import functools

import jax
import jax.numpy as jnp
from jax import lax
from jax.experimental import pallas as pl
from jax.experimental.pallas import tpu as pltpu
from jax.experimental.pallas import tpu_sc as plsc

N = 10000
NP = 10240
E = 640000
G = 128
F_IN = 114
H1 = 230
H1P = 256
H2 = 115
D = 128

NC = 2
NS = 16
NW = NC * NS
EPW = E // NW
CHUNK = 125
NCHUNK = EPW // CHUNK
SEG = 40
NSEG = NCHUNK // SEG
RPS = NP // NS

ROWB = 640
GRID = NP // ROWB


def _sc_mesh():
    return plsc.VectorSubcoreMesh(core_axis_name="c", subcore_axis_name="s")


def _sc_degree(dst3, zeros_np):
    @functools.partial(
        pl.kernel,
        out_type=jax.ShapeDtypeStruct((NC, NP), jnp.float32),
        mesh=_sc_mesh(),
        scratch_types=[
            pltpu.VMEM_SHARED((NP,), jnp.float32),
            pltpu.VMEM((SEG, CHUNK), jnp.int32),
            pltpu.VMEM((128,), jnp.float32),
            pltpu.SemaphoreType.DMA,
            pltpu.SemaphoreType.DMA,
        ],
    )
    def deg_kernel(dst_hbm, zd_hbm, out_hbm, acc, dstv, onesv, sem0, sem1):
        c = lax.axis_index("c")
        s = lax.axis_index("s")
        w = c * NS + s
        pltpu.sync_copy(zd_hbm.at[pl.ds(s * RPS, RPS)],
                        acc.at[pl.ds(s * RPS, RPS)])
        for i in range(128 // 16):
            onesv[pl.ds(i * 16, 16)] = jnp.ones((16,), jnp.float32)
        plsc.subcore_barrier()
        ones_c = onesv.at[pl.ds(0, CHUNK)]

        def seg_body(g, carry):
            pltpu.sync_copy(dst_hbm.at[w * NSEG + g], dstv)

            def body(k2, carry2):
                b0 = 2 * k2
                pltpu.async_copy(ones_c, acc.at[dstv.at[b0]], sem0,
                                 add=True)
                pltpu.async_copy(ones_c, acc.at[dstv.at[b0 + 1]], sem1,
                                 add=True)
                pltpu.make_async_copy(ones_c, acc.at[dstv.at[b0]],
                                      sem0).wait()
                pltpu.make_async_copy(ones_c, acc.at[dstv.at[b0 + 1]],
                                      sem1).wait()
                return carry2

            lax.fori_loop(0, SEG // 2, body, 0)
            return carry

        lax.fori_loop(0, NSEG, seg_body, 0)
        plsc.subcore_barrier()
        pltpu.sync_copy(acc.at[pl.ds(s * RPS, RPS)],
                        out_hbm.at[c, pl.ds(s * RPS, RPS)])

    return deg_kernel(dst3, zeros_np)


def _sc_aggregate(table, src3, dst3, zeros_npd):
    @functools.partial(
        pl.kernel,
        out_type=jax.ShapeDtypeStruct((NC, NP, D), jnp.float32),
        mesh=_sc_mesh(),
        scratch_types=[
            pltpu.VMEM_SHARED((NP, D), jnp.float32),
            pltpu.VMEM((SEG, CHUNK), jnp.int32),
            pltpu.VMEM((SEG, CHUNK), jnp.int32),
            pltpu.VMEM((CHUNK, D), jnp.float32),
            pltpu.VMEM((CHUNK, D), jnp.float32),
            pltpu.SemaphoreType.DMA,
            pltpu.SemaphoreType.DMA,
        ],
    )
    def agg_kernel(table_hbm, src_hbm, dst_hbm, zt_hbm, out_hbm,
                   acc, srcv, dstv, rows0, rows1, sem0, sem1):
        c = lax.axis_index("c")
        s = lax.axis_index("s")
        w = c * NS + s
        pltpu.sync_copy(zt_hbm.at[pl.ds(s * RPS, RPS)],
                        acc.at[pl.ds(s * RPS, RPS)])
        plsc.subcore_barrier()

        def seg_body(g, carry):
            pltpu.sync_copy(src_hbm.at[w * NSEG + g], srcv)
            pltpu.sync_copy(dst_hbm.at[w * NSEG + g], dstv)
            pltpu.async_copy(table_hbm.at[srcv.at[0]], rows0, sem0)

            def body(k2, carry2):
                b0 = 2 * k2
                pltpu.make_async_copy(table_hbm.at[srcv.at[b0]],
                                      rows0, sem0).wait()
                pltpu.async_copy(table_hbm.at[srcv.at[b0 + 1]], rows1, sem1)
                pltpu.sync_copy(rows0, acc.at[dstv.at[b0]], add=True)

                @pl.when(k2 < SEG // 2 - 1)
                def _():
                    pltpu.async_copy(table_hbm.at[srcv.at[b0 + 2]],
                                     rows0, sem0)

                pltpu.make_async_copy(table_hbm.at[srcv.at[b0 + 1]],
                                      rows1, sem1).wait()
                pltpu.sync_copy(rows1, acc.at[dstv.at[b0 + 1]], add=True)
                return carry2

            lax.fori_loop(0, SEG // 2, body, 0)
            return carry

        lax.fori_loop(0, NSEG, seg_body, 0)
        plsc.subcore_barrier()
        pltpu.sync_copy(acc.at[pl.ds(s * RPS, RPS)],
                        out_hbm.at[c, pl.ds(s * RPS, RPS)])

    return agg_kernel(table, src3, dst3, zeros_npd)


GP = G + 8
RPW = NP // NW


def _sc_pool(z, batchp):
    @functools.partial(
        pl.kernel,
        out_type=jax.ShapeDtypeStruct((NW, GP, D), jnp.float32),
        mesh=_sc_mesh(),
        scratch_types=[
            pltpu.VMEM((GP, D), jnp.float32),
            pltpu.VMEM((RPW, D), jnp.float32),
            pltpu.VMEM((RPW + 16,), jnp.int32),
        ],
    )
    def pool_kernel(z_hbm, b_hbm, out_hbm, partial, zv, bv):
        c = lax.axis_index("c")
        s = lax.axis_index("s")
        w = c * NS + s
        pltpu.sync_copy(z_hbm.at[pl.ds(w * RPW, RPW)], zv)
        pltpu.sync_copy(b_hbm.at[pl.ds(w * RPW, RPW)],
                        bv.at[pl.ds(0, RPW)])
        ninf = jnp.full((16,), -jnp.inf, jnp.float32)

        def init_body(i, carry):
            for j in range(D // 16):
                partial[i, pl.ds(16 * j, 16)] = ninf
            return carry

        lax.fori_loop(0, GP, init_body, 0)

        def row_body(i, carry):
            b = bv[pl.ds(i, 16)][0]
            for j in range(D // 16):
                sl = pl.ds(16 * j, 16)
                partial[b, sl] = jnp.maximum(partial[b, sl], zv[i, sl])
            return carry

        lax.fori_loop(0, RPW, row_body, 0)
        pltpu.sync_copy(partial, out_hbm.at[w])

    return pool_kernel(z, batchp)


def _tc_scale(degT):
    def body(deg_ref, dinv_ref):
        d = 1.0 + deg_ref[:, 0:1] + deg_ref[:, 1:2]
        y = lax.rsqrt(d)
        dinv = y * (1.5 - 0.5 * d * y * y)
        dinv_ref[...] = dinv

    return pl.pallas_call(
        body,
        grid=(GRID,),
        in_specs=[
            pl.BlockSpec((ROWB, 2), lambda i: (i, 0)),
        ],
        out_specs=pl.BlockSpec((ROWB, 1), lambda i: (i, 0)),
        out_shape=jax.ShapeDtypeStruct((NP, 1), jnp.float32),
    )(degT)


def _tc_mm1(xp, dinvc, W1p):
    def body(x_ref, dinv_ref, w1_ref, o_ref):
        h = jnp.dot(x_ref[...], w1_ref[...],
                    preferred_element_type=jnp.float32)
        o_ref[...] = dinv_ref[...] * h

    return pl.pallas_call(
        body,
        grid=(GRID,),
        in_specs=[
            pl.BlockSpec((ROWB, D), lambda i: (i, 0)),
            pl.BlockSpec((ROWB, 1), lambda i: (i, 0)),
            pl.BlockSpec((D, H1P), lambda i: (0, 0)),
        ],
        out_specs=pl.BlockSpec((ROWB, H1P), lambda i: (i, 0)),
        out_shape=jax.ShapeDtypeStruct((NP, H1P), jnp.float32),
    )(xp, dinvc, W1p)


def _tc_l1l2(hL, hR, aL0, aL1, aR0, aR1, dinvc, b1p, W2p):
    def body(hl_ref, hr_ref, al0_ref, al1_ref, ar0_ref, ar1_ref,
             dinv_ref, b1_ref, w2_ref, o_ref):
        dinv = dinv_ref[...]
        u = dinv * (hl_ref[...] + al0_ref[...] + al1_ref[...])
        v = dinv * (hr_ref[...] + ar0_ref[...] + ar1_ref[...])
        h1 = jnp.maximum(jnp.concatenate([u, v], axis=1) + b1_ref[...], 0.0)
        m2 = jnp.dot(h1, w2_ref[...], preferred_element_type=jnp.float32)
        o_ref[...] = dinv * m2

    return pl.pallas_call(
        body,
        grid=(GRID,),
        in_specs=[
            pl.BlockSpec((ROWB, D), lambda i: (i, 0)),
            pl.BlockSpec((ROWB, D), lambda i: (i, 0)),
            pl.BlockSpec((ROWB, D), lambda i: (i, 0)),
            pl.BlockSpec((ROWB, D), lambda i: (i, 0)),
            pl.BlockSpec((ROWB, D), lambda i: (i, 0)),
            pl.BlockSpec((ROWB, D), lambda i: (i, 0)),
            pl.BlockSpec((ROWB, 1), lambda i: (i, 0)),
            pl.BlockSpec((1, H1P), lambda i: (0, 0)),
            pl.BlockSpec((H1P, D), lambda i: (0, 0)),
        ],
        out_specs=pl.BlockSpec((ROWB, D), lambda i: (i, 0)),
        out_shape=jax.ShapeDtypeStruct((NP, D), jnp.float32),
    )(hL, hR, aL0, aL1, aR0, aR1, dinvc, b1p, W2p)


def _tc_z(xs2, a0, a1, dinvc):
    def body(x_ref, a0_ref, a1_ref, dinv_ref, o_ref):
        o_ref[...] = dinv_ref[...] * (x_ref[...] + a0_ref[...] + a1_ref[...])

    return pl.pallas_call(
        body,
        grid=(GRID,),
        in_specs=[
            pl.BlockSpec((ROWB, D), lambda i: (i, 0)),
            pl.BlockSpec((ROWB, D), lambda i: (i, 0)),
            pl.BlockSpec((ROWB, D), lambda i: (i, 0)),
            pl.BlockSpec((ROWB, 1), lambda i: (i, 0)),
        ],
        out_specs=pl.BlockSpec((ROWB, D), lambda i: (i, 0)),
        out_shape=jax.ShapeDtypeStruct((NP, D), jnp.float32),
    )(xs2, a0, a1, dinvc)


def _tc_head(parts, b2p, Wgp, bgp, Wfp, bfp, Wop, bop):
    def body(p_ref, b2_ref, wg_ref, bg_ref, wf_ref, bf_ref, wo_ref, bo_ref,
             o_ref):
        pooled = p_ref[0]
        for k in range(1, NW):
            pooled = jnp.maximum(pooled, p_ref[k])
        h = jnp.maximum(pooled + b2_ref[...], 0.0)
        g = jnp.dot(h, wg_ref[...], preferred_element_type=jnp.float32)
        g = jnp.maximum(g + bg_ref[...], 0.0)
        g = jnp.dot(g, wf_ref[...], preferred_element_type=jnp.float32)
        g = jnp.maximum(g + bf_ref[...], 0.0)
        o = jnp.dot(g, wo_ref[...], preferred_element_type=jnp.float32)
        o_ref[...] = o + bo_ref[...]

    return pl.pallas_call(
        body,
        out_shape=jax.ShapeDtypeStruct((G, D), jnp.float32),
    )(parts, b2p, Wgp, bgp, Wfp, bfp, Wop, bop)


def kernel(x, edge_index, batch, W1, b1, W2, b2, Wg, bg, Wf, bf, Wo, bo):
    f32 = jnp.float32
    src = edge_index[0].astype(jnp.int32)
    dst = edge_index[1].astype(jnp.int32)
    src3 = src.reshape(NW * NSEG, SEG, CHUNK)
    dst3 = dst.reshape(NW * NSEG, SEG, CHUNK)

    xp = jnp.pad(x.astype(f32), ((0, NP - N), (0, D - F_IN)))
    zt = jnp.zeros((NP, D), f32)
    zd = jnp.zeros((NP,), f32)

    W1p = jnp.pad(W1, ((0, D - F_IN), (0, H1P - H1)))
    b1p = jnp.pad(b1, (0, H1P - H1)).reshape(1, H1P)
    W2p = jnp.pad(W2, ((0, H1P - H1), (0, D - H2)))
    b2p = jnp.pad(b2, (0, D - H2)).reshape(1, D)
    Wgp = jnp.pad(Wg, ((0, D - H2), (0, 128 - 64)))
    bgp = jnp.pad(bg, (0, 128 - 64)).reshape(1, 128)
    Wfp = jnp.pad(Wf, ((0, 128 - 64), (0, 128 - 32)))
    bfp = jnp.pad(bf, (0, 128 - 32)).reshape(1, 128)
    Wop = jnp.pad(Wo, ((0, 128 - 32), (0, 128 - 1)))
    bop = jnp.pad(bo, (0, 128 - 1)).reshape(1, 128)

    deg2 = _sc_degree(dst3, zd)
    degT = jnp.transpose(deg2)
    dinvc = _tc_scale(degT)

    hs = _tc_mm1(xp, dinvc, W1p)
    hL = hs[:, :D]
    hR = hs[:, D:]
    aggL = _sc_aggregate(hL, src3, dst3, zt)
    aggR = _sc_aggregate(hR, src3, dst3, zt)
    xs2 = _tc_l1l2(hL, hR, aggL[0], aggL[1], aggR[0], aggR[1],
                   dinvc, b1p, W2p)

    agg2 = _sc_aggregate(xs2, src3, dst3, zt)
    z = _tc_z(xs2, agg2[0], agg2[1], dinvc)

    batchp = jnp.pad(batch.astype(jnp.int32), (0, NP - N),
                     constant_values=G)
    partials = _sc_pool(z, batchp)
    parts = partials[:, :G, :]
    outf = _tc_head(parts, b2p, Wgp, bgp, Wfp, bfp, Wop, bop)
    return outf[:, :1]


# speedup vs baseline: 26.6913x; 1.0091x over previous
import functools

import jax
import jax.numpy as jnp
from jax import lax
from jax.experimental import pallas as pl
from jax.experimental.pallas import tpu as pltpu
from jax.experimental.pallas import tpu_sc as plsc

N = 10000
NP = 10240
E = 640000
G = 128
F_IN = 114
H1 = 230
H1P = 256
H2 = 115
D = 128

NC = 2
NS = 16
NW = NC * NS
EPW = E // NW
CHUNK = 125
NCHUNK = EPW // CHUNK
SEG = 40
NSEG = NCHUNK // SEG
RPS = NP // NS

ROWB = 640
GRID = NP // ROWB
WR = D
KP = D + WR


def _sc_mesh():
    return plsc.VectorSubcoreMesh(core_axis_name="c", subcore_axis_name="s")


def _sc_degree(dst3, zeros_np):
    @functools.partial(
        pl.kernel,
        out_type=jax.ShapeDtypeStruct((NC, NP), jnp.float32),
        mesh=_sc_mesh(),
        scratch_types=[
            pltpu.VMEM_SHARED((NP,), jnp.float32),
            pltpu.VMEM((SEG, CHUNK), jnp.int32),
            pltpu.VMEM((128,), jnp.float32),
            pltpu.SemaphoreType.DMA,
            pltpu.SemaphoreType.DMA,
        ],
    )
    def deg_kernel(dst_hbm, zd_hbm, out_hbm, acc, dstv, onesv, sem0, sem1):
        c = lax.axis_index("c")
        s = lax.axis_index("s")
        w = c * NS + s
        pltpu.sync_copy(zd_hbm.at[pl.ds(s * RPS, RPS)],
                        acc.at[pl.ds(s * RPS, RPS)])
        for i in range(128 // 16):
            onesv[pl.ds(i * 16, 16)] = jnp.ones((16,), jnp.float32)
        plsc.subcore_barrier()
        ones_c = onesv.at[pl.ds(0, CHUNK)]

        def seg_body(g, carry):
            pltpu.sync_copy(dst_hbm.at[w * NSEG + g], dstv)

            def body(k2, carry2):
                b0 = 2 * k2
                pltpu.async_copy(ones_c, acc.at[dstv.at[b0]], sem0,
                                 add=True)
                pltpu.async_copy(ones_c, acc.at[dstv.at[b0 + 1]], sem1,
                                 add=True)
                pltpu.make_async_copy(ones_c, acc.at[dstv.at[b0]],
                                      sem0).wait()
                pltpu.make_async_copy(ones_c, acc.at[dstv.at[b0 + 1]],
                                      sem1).wait()
                return carry2

            lax.fori_loop(0, SEG // 2, body, 0)
            return carry

        lax.fori_loop(0, NSEG, seg_body, 0)
        plsc.subcore_barrier()
        pltpu.sync_copy(acc.at[pl.ds(s * RPS, RPS)],
                        out_hbm.at[c, pl.ds(s * RPS, RPS)])

    return deg_kernel(dst3, zeros_np)


def _sc_aggregate(table, src3, dst3, zeros_npd, width=D):
    @functools.partial(
        pl.kernel,
        out_type=jax.ShapeDtypeStruct((NC, NP, width), jnp.float32),
        mesh=_sc_mesh(),
        scratch_types=[
            pltpu.VMEM_SHARED((NP, width), jnp.float32),
            pltpu.VMEM((SEG, CHUNK), jnp.int32),
            pltpu.VMEM((SEG, CHUNK), jnp.int32),
            pltpu.VMEM((CHUNK, width), jnp.float32),
            pltpu.VMEM((CHUNK, width), jnp.float32),
            pltpu.SemaphoreType.DMA,
            pltpu.SemaphoreType.DMA,
        ],
    )
    def agg_kernel(table_hbm, src_hbm, dst_hbm, zt_hbm, out_hbm,
                   acc, srcv, dstv, rows0, rows1, sem0, sem1):
        c = lax.axis_index("c")
        s = lax.axis_index("s")
        w = c * NS + s
        pltpu.sync_copy(zt_hbm.at[pl.ds(s * RPS, RPS)],
                        acc.at[pl.ds(s * RPS, RPS)])
        plsc.subcore_barrier()

        def seg_body(g, carry):
            pltpu.sync_copy(src_hbm.at[w * NSEG + g], srcv)
            pltpu.sync_copy(dst_hbm.at[w * NSEG + g], dstv)
            pltpu.async_copy(table_hbm.at[srcv.at[0]], rows0, sem0)

            def body(k2, carry2):
                b0 = 2 * k2
                pltpu.make_async_copy(table_hbm.at[srcv.at[b0]],
                                      rows0, sem0).wait()
                pltpu.async_copy(table_hbm.at[srcv.at[b0 + 1]], rows1, sem1)
                pltpu.sync_copy(rows0, acc.at[dstv.at[b0]], add=True)

                @pl.when(k2 < SEG // 2 - 1)
                def _():
                    pltpu.async_copy(table_hbm.at[srcv.at[b0 + 2]],
                                     rows0, sem0)

                pltpu.make_async_copy(table_hbm.at[srcv.at[b0 + 1]],
                                      rows1, sem1).wait()
                pltpu.sync_copy(rows1, acc.at[dstv.at[b0 + 1]], add=True)
                return carry2

            lax.fori_loop(0, SEG // 2, body, 0)
            return carry

        lax.fori_loop(0, NSEG, seg_body, 0)
        plsc.subcore_barrier()
        pltpu.sync_copy(acc.at[pl.ds(s * RPS, RPS)],
                        out_hbm.at[c, pl.ds(s * RPS, RPS)])

    return agg_kernel(table, src3, dst3, zeros_npd)


GP = G + 8
RPW = NP // NW


def _sc_pool(z, batchp):
    @functools.partial(
        pl.kernel,
        out_type=jax.ShapeDtypeStruct((NW, GP, D), jnp.float32),
        mesh=_sc_mesh(),
        scratch_types=[
            pltpu.VMEM((GP, D), jnp.float32),
            pltpu.VMEM((RPW, D), jnp.float32),
            pltpu.VMEM((RPW + 16,), jnp.int32),
        ],
    )
    def pool_kernel(z_hbm, b_hbm, out_hbm, partial, zv, bv):
        c = lax.axis_index("c")
        s = lax.axis_index("s")
        w = c * NS + s
        pltpu.sync_copy(z_hbm.at[pl.ds(w * RPW, RPW)], zv)
        pltpu.sync_copy(b_hbm.at[pl.ds(w * RPW, RPW)],
                        bv.at[pl.ds(0, RPW)])
        ninf = jnp.full((16,), -jnp.inf, jnp.float32)

        def init_body(i, carry):
            for j in range(D // 16):
                partial[i, pl.ds(16 * j, 16)] = ninf
            return carry

        lax.fori_loop(0, GP, init_body, 0)

        def row_body(i, carry):
            b = bv[pl.ds(i, 16)][0]
            for j in range(D // 16):
                sl = pl.ds(16 * j, 16)
                partial[b, sl] = jnp.maximum(partial[b, sl], zv[i, sl])
            return carry

        lax.fori_loop(0, RPW, row_body, 0)
        pltpu.sync_copy(partial, out_hbm.at[w])

    return pool_kernel(z, batchp)


def _tc_mm1(degT, xp, W1p):
    def body(deg_ref, x_ref, w1_ref, dinv_ref, o_ref):
        d = 1.0 + deg_ref[:, 0:1] + deg_ref[:, 1:2]
        y = lax.rsqrt(d)
        dinv = y * (1.5 - 0.5 * d * y * y)
        dinv_ref[...] = dinv
        h = jnp.dot(x_ref[...], w1_ref[...],
                    preferred_element_type=jnp.float32)
        o_ref[...] = dinv * h

    return pl.pallas_call(
        body,
        grid=(GRID,),
        in_specs=[
            pl.BlockSpec((ROWB, 2), lambda i: (i, 0)),
            pl.BlockSpec((ROWB, D), lambda i: (i, 0)),
            pl.BlockSpec((D, H1P), lambda i: (0, 0)),
        ],
        out_specs=[
            pl.BlockSpec((ROWB, 1), lambda i: (i, 0)),
            pl.BlockSpec((ROWB, H1P), lambda i: (i, 0)),
        ],
        out_shape=[
            jax.ShapeDtypeStruct((NP, 1), jnp.float32),
            jax.ShapeDtypeStruct((NP, H1P), jnp.float32),
        ],
    )(degT, xp, W1p)


def _tc_l1l2(hL, hR, aL0, aL1, aR0, aR1, dinvc, b1p, W2p):
    def body(hl_ref, hr_ref, al0_ref, al1_ref, ar0_ref, ar1_ref,
             dinv_ref, b1_ref, w2_ref, o_ref):
        dinv = dinv_ref[...]
        u = dinv * (hl_ref[...] + al0_ref[...] + al1_ref[...])
        v = dinv * (hr_ref[...] + ar0_ref[...] + ar1_ref[...])
        h1 = jnp.maximum(jnp.concatenate([u, v], axis=1) + b1_ref[...], 0.0)
        m2 = jnp.dot(h1, w2_ref[...], preferred_element_type=jnp.float32)
        o_ref[...] = dinv * m2

    return pl.pallas_call(
        body,
        grid=(GRID,),
        in_specs=[
            pl.BlockSpec((ROWB, D), lambda i: (i, 0)),
            pl.BlockSpec((ROWB, WR), lambda i: (i, 0)),
            pl.BlockSpec((ROWB, D), lambda i: (i, 0)),
            pl.BlockSpec((ROWB, D), lambda i: (i, 0)),
            pl.BlockSpec((ROWB, WR), lambda i: (i, 0)),
            pl.BlockSpec((ROWB, WR), lambda i: (i, 0)),
            pl.BlockSpec((ROWB, 1), lambda i: (i, 0)),
            pl.BlockSpec((1, KP), lambda i: (0, 0)),
            pl.BlockSpec((KP, D), lambda i: (0, 0)),
        ],
        out_specs=pl.BlockSpec((ROWB, D), lambda i: (i, 0)),
        out_shape=jax.ShapeDtypeStruct((NP, D), jnp.float32),
    )(hL, hR, aL0, aL1, aR0, aR1, dinvc, b1p, W2p)


def _tc_z(xs2, a0, a1, dinvc):
    def body(x_ref, a0_ref, a1_ref, dinv_ref, o_ref):
        o_ref[...] = dinv_ref[...] * (x_ref[...] + a0_ref[...] + a1_ref[...])

    return pl.pallas_call(
        body,
        grid=(GRID,),
        in_specs=[
            pl.BlockSpec((ROWB, D), lambda i: (i, 0)),
            pl.BlockSpec((ROWB, D), lambda i: (i, 0)),
            pl.BlockSpec((ROWB, D), lambda i: (i, 0)),
            pl.BlockSpec((ROWB, 1), lambda i: (i, 0)),
        ],
        out_specs=pl.BlockSpec((ROWB, D), lambda i: (i, 0)),
        out_shape=jax.ShapeDtypeStruct((NP, D), jnp.float32),
    )(xs2, a0, a1, dinvc)


def _tc_head(parts, b2p, Wgp, bgp, Wfp, bfp, Wop, bop):
    def body(p_ref, b2_ref, wg_ref, bg_ref, wf_ref, bf_ref, wo_ref, bo_ref,
             o_ref):
        pooled = p_ref[0]
        for k in range(1, NW):
            pooled = jnp.maximum(pooled, p_ref[k])
        h = jnp.maximum(pooled + b2_ref[...], 0.0)
        g = jnp.dot(h, wg_ref[...], preferred_element_type=jnp.float32)
        g = jnp.maximum(g + bg_ref[...], 0.0)
        g = jnp.dot(g, wf_ref[...], preferred_element_type=jnp.float32)
        g = jnp.maximum(g + bf_ref[...], 0.0)
        o = jnp.dot(g, wo_ref[...], preferred_element_type=jnp.float32)
        o_ref[...] = o + bo_ref[...]

    return pl.pallas_call(
        body,
        out_shape=jax.ShapeDtypeStruct((G, D), jnp.float32),
    )(parts, b2p, Wgp, bgp, Wfp, bfp, Wop, bop)


def kernel(x, edge_index, batch, W1, b1, W2, b2, Wg, bg, Wf, bf, Wo, bo):
    f32 = jnp.float32
    src = edge_index[0].astype(jnp.int32)
    dst = edge_index[1].astype(jnp.int32)
    src3 = src.reshape(NW * NSEG, SEG, CHUNK)
    dst3 = dst.reshape(NW * NSEG, SEG, CHUNK)

    xp = jnp.pad(x.astype(f32), ((0, NP - N), (0, D - F_IN)))
    zt = jnp.zeros((NP, D), f32)
    zd = jnp.zeros((NP,), f32)

    W1p = jnp.pad(W1, ((0, D - F_IN), (0, H1P - H1)))
    b1p = jnp.pad(b1, (0, KP - H1)).reshape(1, KP)
    W2p = jnp.pad(W2, ((0, KP - H1), (0, D - H2)))
    b2p = jnp.pad(b2, (0, D - H2)).reshape(1, D)
    Wgp = jnp.pad(Wg, ((0, D - H2), (0, 128 - 64)))
    bgp = jnp.pad(bg, (0, 128 - 64)).reshape(1, 128)
    Wfp = jnp.pad(Wf, ((0, 128 - 64), (0, 128 - 32)))
    bfp = jnp.pad(bf, (0, 128 - 32)).reshape(1, 128)
    Wop = jnp.pad(Wo, ((0, 128 - 32), (0, 128 - 1)))
    bop = jnp.pad(bo, (0, 128 - 1)).reshape(1, 128)

    deg2 = _sc_degree(dst3, zd)
    degT = jnp.transpose(deg2)

    dinvc, hs = _tc_mm1(degT, xp, W1p)
    hL = hs[:, :D]
    hR = hs[:, D:KP]
    aggL = _sc_aggregate(hL, src3, dst3, zt)
    aggR = _sc_aggregate(hR, src3, dst3, zt)
    xs2 = _tc_l1l2(hL, hR, aggL[0], aggL[1], aggR[0], aggR[1],
                   dinvc, b1p, W2p)

    agg2 = _sc_aggregate(xs2, src3, dst3, zt)
    z = _tc_z(xs2, agg2[0], agg2[1], dinvc)

    batchp = jnp.pad(batch.astype(jnp.int32), (0, NP - N),
                     constant_values=G)
    partials = _sc_pool(z, batchp)
    parts = partials[:, :G, :]
    outf = _tc_head(parts, b2p, Wgp, bgp, Wfp, bfp, Wop, bop)
    return outf[:, :1]


# speedup vs baseline: 27.5394x; 1.0318x over previous
import functools

import jax
import jax.numpy as jnp
from jax import lax
from jax.experimental import pallas as pl
from jax.experimental.pallas import tpu as pltpu
from jax.experimental.pallas import tpu_sc as plsc

N = 10000
NP = 10240
E = 640000
G = 128
F_IN = 114
H1 = 230
H1P = 256
H2 = 115
D = 128

NC = 2
NS = 16
NW = NC * NS
EPW = E // NW
CHUNK = 125
NCHUNK = EPW // CHUNK
SEG = 40
NSEG = NCHUNK // SEG
RPS = NP // NS

ROWB = 640
GRID = NP // ROWB
WR = D
KP = D + WR


def _sc_mesh():
    return plsc.VectorSubcoreMesh(core_axis_name="c", subcore_axis_name="s")


def _sc_degree(dst3, zeros_np):
    @functools.partial(
        pl.kernel,
        out_type=jax.ShapeDtypeStruct((NC, NP), jnp.float32),
        mesh=_sc_mesh(),
        scratch_types=[
            pltpu.VMEM_SHARED((NP,), jnp.float32),
            pltpu.VMEM((SEG, CHUNK), jnp.int32),
            pltpu.VMEM((128,), jnp.float32),
            pltpu.SemaphoreType.DMA,
            pltpu.SemaphoreType.DMA,
        ],
    )
    def deg_kernel(dst_hbm, zd_hbm, out_hbm, acc, dstv, onesv, sem0, sem1):
        c = lax.axis_index("c")
        s = lax.axis_index("s")
        w = c * NS + s
        pltpu.sync_copy(zd_hbm.at[pl.ds(s * RPS, RPS)],
                        acc.at[pl.ds(s * RPS, RPS)])
        for i in range(128 // 16):
            onesv[pl.ds(i * 16, 16)] = jnp.ones((16,), jnp.float32)
        plsc.subcore_barrier()
        ones_c = onesv.at[pl.ds(0, CHUNK)]

        def seg_body(g, carry):
            pltpu.sync_copy(dst_hbm.at[w * NSEG + g], dstv)

            def body(k2, carry2):
                b0 = 2 * k2
                pltpu.async_copy(ones_c, acc.at[dstv.at[b0]], sem0,
                                 add=True)
                pltpu.async_copy(ones_c, acc.at[dstv.at[b0 + 1]], sem1,
                                 add=True)
                pltpu.make_async_copy(ones_c, acc.at[dstv.at[b0]],
                                      sem0).wait()
                pltpu.make_async_copy(ones_c, acc.at[dstv.at[b0 + 1]],
                                      sem1).wait()
                return carry2

            lax.fori_loop(0, SEG // 2, body, 0)
            return carry

        lax.fori_loop(0, NSEG, seg_body, 0)
        plsc.subcore_barrier()
        pltpu.sync_copy(acc.at[pl.ds(s * RPS, RPS)],
                        out_hbm.at[c, pl.ds(s * RPS, RPS)])

    return deg_kernel(dst3, zeros_np)


def _sc_aggregate(table, src3, dst3, zeros_npd, width=D):
    @functools.partial(
        pl.kernel,
        out_type=jax.ShapeDtypeStruct((NC, NP, width), jnp.float32),
        mesh=_sc_mesh(),
        scratch_types=[
            pltpu.VMEM_SHARED((NP, width), jnp.float32),
            pltpu.VMEM((SEG, CHUNK), jnp.int32),
            pltpu.VMEM((SEG, CHUNK), jnp.int32),
            pltpu.VMEM((CHUNK, width), jnp.float32),
            pltpu.VMEM((CHUNK, width), jnp.float32),
            pltpu.SemaphoreType.DMA,
            pltpu.SemaphoreType.DMA,
        ],
    )
    def agg_kernel(table_hbm, src_hbm, dst_hbm, zt_hbm, out_hbm,
                   acc, srcv, dstv, rows0, rows1, sem0, sem1):
        c = lax.axis_index("c")
        s = lax.axis_index("s")
        w = c * NS + s
        pltpu.sync_copy(zt_hbm.at[pl.ds(s * RPS, RPS)],
                        acc.at[pl.ds(s * RPS, RPS)])
        plsc.subcore_barrier()

        def seg_body(g, carry):
            pltpu.sync_copy(src_hbm.at[w * NSEG + g], srcv)
            pltpu.sync_copy(dst_hbm.at[w * NSEG + g], dstv)
            pltpu.async_copy(table_hbm.at[srcv.at[0]], rows0, sem0)

            def body(k2, carry2):
                b0 = 2 * k2
                pltpu.make_async_copy(table_hbm.at[srcv.at[b0]],
                                      rows0, sem0).wait()
                pltpu.async_copy(table_hbm.at[srcv.at[b0 + 1]], rows1, sem1)
                pltpu.sync_copy(rows0, acc.at[dstv.at[b0]], add=True)

                @pl.when(k2 < SEG // 2 - 1)
                def _():
                    pltpu.async_copy(table_hbm.at[srcv.at[b0 + 2]],
                                     rows0, sem0)

                pltpu.make_async_copy(table_hbm.at[srcv.at[b0 + 1]],
                                      rows1, sem1).wait()
                pltpu.sync_copy(rows1, acc.at[dstv.at[b0 + 1]], add=True)
                return carry2

            lax.fori_loop(0, SEG // 2, body, 0)
            return carry

        lax.fori_loop(0, NSEG, seg_body, 0)
        plsc.subcore_barrier()
        pltpu.sync_copy(acc.at[pl.ds(s * RPS, RPS)],
                        out_hbm.at[c, pl.ds(s * RPS, RPS)])

    return agg_kernel(table, src3, dst3, zeros_npd)


EPW2 = E // NS
NSEG2 = EPW2 // CHUNK // SEG


def _sc_aggregate_cols(tableLR, src3, dst3, zeros_npd):
    @functools.partial(
        pl.kernel,
        out_type=jax.ShapeDtypeStruct((NC, NP, D), jnp.float32),
        mesh=_sc_mesh(),
        scratch_types=[
            pltpu.VMEM_SHARED((NP, D), jnp.float32),
            pltpu.VMEM((SEG, CHUNK), jnp.int32),
            pltpu.VMEM((SEG, CHUNK), jnp.int32),
            pltpu.VMEM((CHUNK, D), jnp.float32),
            pltpu.VMEM((CHUNK, D), jnp.float32),
            pltpu.SemaphoreType.DMA,
            pltpu.SemaphoreType.DMA,
        ],
    )
    def aggc_kernel(table_hbm, src_hbm, dst_hbm, zt_hbm, out_hbm,
                    acc, srcv, dstv, rows0, rows1, sem0, sem1):
        c = lax.axis_index("c")
        s = lax.axis_index("s")
        pltpu.sync_copy(zt_hbm.at[pl.ds(s * RPS, RPS)],
                        acc.at[pl.ds(s * RPS, RPS)])
        plsc.subcore_barrier()
        half = table_hbm.at[c]

        def seg_body(g, carry):
            pltpu.sync_copy(src_hbm.at[s * NSEG2 + g], srcv)
            pltpu.sync_copy(dst_hbm.at[s * NSEG2 + g], dstv)
            pltpu.async_copy(half.at[srcv.at[0]], rows0, sem0)

            def body(k2, carry2):
                b0 = 2 * k2
                pltpu.make_async_copy(half.at[srcv.at[b0]],
                                      rows0, sem0).wait()
                pltpu.async_copy(half.at[srcv.at[b0 + 1]], rows1, sem1)
                pltpu.sync_copy(rows0, acc.at[dstv.at[b0]], add=True)

                @pl.when(k2 < SEG // 2 - 1)
                def _():
                    pltpu.async_copy(half.at[srcv.at[b0 + 2]],
                                     rows0, sem0)

                pltpu.make_async_copy(half.at[srcv.at[b0 + 1]],
                                      rows1, sem1).wait()
                pltpu.sync_copy(rows1, acc.at[dstv.at[b0 + 1]], add=True)
                return carry2

            lax.fori_loop(0, SEG // 2, body, 0)
            return carry

        lax.fori_loop(0, NSEG2, seg_body, 0)
        plsc.subcore_barrier()
        pltpu.sync_copy(acc.at[pl.ds(s * RPS, RPS)],
                        out_hbm.at[c, pl.ds(s * RPS, RPS)])

    return aggc_kernel(tableLR, src3, dst3, zeros_npd)


GP = G + 8
RPW = NP // NW


def _sc_pool(z, batchp):
    @functools.partial(
        pl.kernel,
        out_type=jax.ShapeDtypeStruct((NW, GP, D), jnp.float32),
        mesh=_sc_mesh(),
        scratch_types=[
            pltpu.VMEM((GP, D), jnp.float32),
            pltpu.VMEM((RPW, D), jnp.float32),
            pltpu.VMEM((RPW + 16,), jnp.int32),
        ],
    )
    def pool_kernel(z_hbm, b_hbm, out_hbm, partial, zv, bv):
        c = lax.axis_index("c")
        s = lax.axis_index("s")
        w = c * NS + s
        pltpu.sync_copy(z_hbm.at[pl.ds(w * RPW, RPW)], zv)
        pltpu.sync_copy(b_hbm.at[pl.ds(w * RPW, RPW)],
                        bv.at[pl.ds(0, RPW)])
        ninf = jnp.full((16,), -jnp.inf, jnp.float32)

        def init_body(i, carry):
            for j in range(D // 16):
                partial[i, pl.ds(16 * j, 16)] = ninf
            return carry

        lax.fori_loop(0, GP, init_body, 0)

        def row_body(i, carry):
            b = bv[pl.ds(i, 16)][0]
            for j in range(D // 16):
                sl = pl.ds(16 * j, 16)
                partial[b, sl] = jnp.maximum(partial[b, sl], zv[i, sl])
            return carry

        lax.fori_loop(0, RPW, row_body, 0)
        pltpu.sync_copy(partial, out_hbm.at[w])

    return pool_kernel(z, batchp)


def _tc_mm1(degT, xp, W1p):
    def body(deg_ref, x_ref, w1_ref, dinv_ref, o_ref):
        d = 1.0 + deg_ref[:, 0:1] + deg_ref[:, 1:2]
        y = lax.rsqrt(d)
        dinv = y * (1.5 - 0.5 * d * y * y)
        dinv_ref[...] = dinv
        h = jnp.dot(x_ref[...], w1_ref[...],
                    preferred_element_type=jnp.float32)
        o_ref[...] = dinv * h

    return pl.pallas_call(
        body,
        grid=(GRID,),
        in_specs=[
            pl.BlockSpec((ROWB, 2), lambda i: (i, 0)),
            pl.BlockSpec((ROWB, D), lambda i: (i, 0)),
            pl.BlockSpec((D, H1P), lambda i: (0, 0)),
        ],
        out_specs=[
            pl.BlockSpec((ROWB, 1), lambda i: (i, 0)),
            pl.BlockSpec((ROWB, H1P), lambda i: (i, 0)),
        ],
        out_shape=[
            jax.ShapeDtypeStruct((NP, 1), jnp.float32),
            jax.ShapeDtypeStruct((NP, H1P), jnp.float32),
        ],
    )(degT, xp, W1p)


def _tc_l1l2(hL, hR, aL, aR, dinvc, b1p, W2p):
    def body(hl_ref, hr_ref, al_ref, ar_ref,
             dinv_ref, b1_ref, w2_ref, o_ref):
        dinv = dinv_ref[...]
        u = dinv * (hl_ref[...] + al_ref[...])
        v = dinv * (hr_ref[...] + ar_ref[...])
        h1 = jnp.maximum(jnp.concatenate([u, v], axis=1) + b1_ref[...], 0.0)
        m2 = jnp.dot(h1, w2_ref[...], preferred_element_type=jnp.float32)
        o_ref[...] = dinv * m2

    return pl.pallas_call(
        body,
        grid=(GRID,),
        in_specs=[
            pl.BlockSpec((ROWB, D), lambda i: (i, 0)),
            pl.BlockSpec((ROWB, WR), lambda i: (i, 0)),
            pl.BlockSpec((ROWB, D), lambda i: (i, 0)),
            pl.BlockSpec((ROWB, WR), lambda i: (i, 0)),
            pl.BlockSpec((ROWB, 1), lambda i: (i, 0)),
            pl.BlockSpec((1, KP), lambda i: (0, 0)),
            pl.BlockSpec((KP, D), lambda i: (0, 0)),
        ],
        out_specs=pl.BlockSpec((ROWB, D), lambda i: (i, 0)),
        out_shape=jax.ShapeDtypeStruct((NP, D), jnp.float32),
    )(hL, hR, aL, aR, dinvc, b1p, W2p)


def _tc_z(xs2, a0, a1, dinvc):
    def body(x_ref, a0_ref, a1_ref, dinv_ref, o_ref):
        o_ref[...] = dinv_ref[...] * (x_ref[...] + a0_ref[...] + a1_ref[...])

    return pl.pallas_call(
        body,
        grid=(GRID,),
        in_specs=[
            pl.BlockSpec((ROWB, D), lambda i: (i, 0)),
            pl.BlockSpec((ROWB, D), lambda i: (i, 0)),
            pl.BlockSpec((ROWB, D), lambda i: (i, 0)),
            pl.BlockSpec((ROWB, 1), lambda i: (i, 0)),
        ],
        out_specs=pl.BlockSpec((ROWB, D), lambda i: (i, 0)),
        out_shape=jax.ShapeDtypeStruct((NP, D), jnp.float32),
    )(xs2, a0, a1, dinvc)


def _tc_head(parts, b2p, Wgp, bgp, Wfp, bfp, Wop, bop):
    def body(p_ref, b2_ref, wg_ref, bg_ref, wf_ref, bf_ref, wo_ref, bo_ref,
             o_ref):
        pooled = p_ref[0]
        for k in range(1, NW):
            pooled = jnp.maximum(pooled, p_ref[k])
        h = jnp.maximum(pooled + b2_ref[...], 0.0)
        g = jnp.dot(h, wg_ref[...], preferred_element_type=jnp.float32)
        g = jnp.maximum(g + bg_ref[...], 0.0)
        g = jnp.dot(g, wf_ref[...], preferred_element_type=jnp.float32)
        g = jnp.maximum(g + bf_ref[...], 0.0)
        o = jnp.dot(g, wo_ref[...], preferred_element_type=jnp.float32)
        o_ref[...] = o + bo_ref[...]

    return pl.pallas_call(
        body,
        out_shape=jax.ShapeDtypeStruct((G, D), jnp.float32),
    )(parts, b2p, Wgp, bgp, Wfp, bfp, Wop, bop)


def kernel(x, edge_index, batch, W1, b1, W2, b2, Wg, bg, Wf, bf, Wo, bo):
    f32 = jnp.float32
    src = edge_index[0].astype(jnp.int32)
    dst = edge_index[1].astype(jnp.int32)
    src3 = src.reshape(NW * NSEG, SEG, CHUNK)
    dst3 = dst.reshape(NW * NSEG, SEG, CHUNK)

    xp = jnp.pad(x.astype(f32), ((0, NP - N), (0, D - F_IN)))
    zt = jnp.zeros((NP, D), f32)
    zd = jnp.zeros((NP,), f32)

    W1p = jnp.pad(W1, ((0, D - F_IN), (0, H1P - H1)))
    b1p = jnp.pad(b1, (0, KP - H1)).reshape(1, KP)
    W2p = jnp.pad(W2, ((0, KP - H1), (0, D - H2)))
    b2p = jnp.pad(b2, (0, D - H2)).reshape(1, D)
    Wgp = jnp.pad(Wg, ((0, D - H2), (0, 128 - 64)))
    bgp = jnp.pad(bg, (0, 128 - 64)).reshape(1, 128)
    Wfp = jnp.pad(Wf, ((0, 128 - 64), (0, 128 - 32)))
    bfp = jnp.pad(bf, (0, 128 - 32)).reshape(1, 128)
    Wop = jnp.pad(Wo, ((0, 128 - 32), (0, 128 - 1)))
    bop = jnp.pad(bo, (0, 128 - 1)).reshape(1, 128)

    deg2 = _sc_degree(dst3, zd)
    degT = jnp.transpose(deg2)

    dinvc, hs = _tc_mm1(degT, xp, W1p)
    hL = hs[:, :D]
    hR = hs[:, D:KP]
    hLR = jnp.stack([hL, hR])
    agg1 = _sc_aggregate_cols(hLR, src3, dst3, zt)
    xs2 = _tc_l1l2(hL, hR, agg1[0], agg1[1],
                   dinvc, b1p, W2p)

    agg2 = _sc_aggregate(xs2, src3, dst3, zt)
    z = _tc_z(xs2, agg2[0], agg2[1], dinvc)

    batchp = jnp.pad(batch.astype(jnp.int32), (0, NP - N),
                     constant_values=G)
    partials = _sc_pool(z, batchp)
    parts = partials[:, :G, :]
    outf = _tc_head(parts, b2p, Wgp, bgp, Wfp, bfp, Wop, bop)
    return outf[:, :1]
